# Initial kernel scaffold; baseline (speedup 1.0000x reference)
import functools

import jax
import jax.numpy as jnp
from jax import lax
from jax.experimental import pallas as pl
from jax.experimental.pallas import tpu as pltpu
from jax.experimental.pallas import tpu_sc as plsc

N_NODES = 10000
D_IN = 128
D_HID = 16
D_OUT = 128

NC = 2
NS = 16
NW = NC * NS
CHUNK = 128
CH = 80
E_PAD = NW * CH * CHUNK

N_PAD = 10240
ROWS_PER_SUB = N_PAD // NS


def _segsum(h, src_p, dst_p):
    mesh = plsc.VectorSubcoreMesh(core_axis_name="c", subcore_axis_name="s")

    @functools.partial(
        pl.kernel,
        mesh=mesh,
        out_type=jax.ShapeDtypeStruct((NC, N_PAD, D_HID), jnp.float32),
        scratch_types=[
            pltpu.VMEM((CH, CHUNK), jnp.int32),
            pltpu.VMEM((CH, CHUNK), jnp.int32),
            pltpu.VMEM((CHUNK, D_HID), jnp.float32),
            pltpu.VMEM((ROWS_PER_SUB, D_HID), jnp.float32),
            pltpu.VMEM_SHARED((N_PAD, D_HID), jnp.float32),
            pltpu.SemaphoreType.DMA,
        ],
    )
    def segsum_kernel(h_hbm, src_hbm, dst_hbm, out_hbm,
                      src_v, dst_v, msg_v, zero_v, acc_sh, gsem):
        c = lax.axis_index("c")
        s = lax.axis_index("s")
        wid = c * NS + s

        zval = jnp.zeros((D_HID,), jnp.float32)

        @pl.loop(0, ROWS_PER_SUB)
        def _zero(i):
            zero_v[i, :] = zval

        pltpu.sync_copy(zero_v, acc_sh.at[pl.ds(s * ROWS_PER_SUB, ROWS_PER_SUB)])
        plsc.subcore_barrier()

        pltpu.sync_copy(src_hbm.at[wid], src_v)
        pltpu.sync_copy(dst_hbm.at[wid], dst_v)

        @pl.loop(0, CH)
        def _edges(j):
            pltpu.async_copy(h_hbm.at[src_v.at[j]], msg_v, gsem).wait()
            pltpu.sync_copy(msg_v, acc_sh.at[dst_v.at[j]], add=True)

        plsc.subcore_barrier()
        pltpu.sync_copy(
            acc_sh.at[pl.ds(s * ROWS_PER_SUB, ROWS_PER_SUB)],
            out_hbm.at[c, pl.ds(s * ROWS_PER_SUB, ROWS_PER_SUB)],
        )

    return segsum_kernel(h, src_p, dst_p)


def _project_in(features, W1):
    def body(f_ref, w_ref, o_ref):
        o_ref[...] = jnp.dot(f_ref[...], w_ref[...],
                             preferred_element_type=jnp.float32)

    return pl.pallas_call(
        body,
        out_shape=jax.ShapeDtypeStruct((N_NODES, D_HID), jnp.float32),
    )(features, W1)


def _bias_relu(p, b1):
    def body(p_ref, b_ref, o_ref):
        o_ref[...] = jnp.maximum(p_ref[0] + p_ref[1] + b_ref[...], 0.0)

    return pl.pallas_call(
        body,
        out_shape=jax.ShapeDtypeStruct((N_NODES, D_HID), jnp.float32),
    )(p, b1.reshape(1, D_HID))


def _project_out(q, W2, b2):
    def body(q_ref, w_ref, b_ref, o_ref):
        agg = q_ref[0] + q_ref[1]
        o_ref[...] = jnp.dot(agg, w_ref[...],
                             preferred_element_type=jnp.float32) + b_ref[...]

    return pl.pallas_call(
        body,
        out_shape=jax.ShapeDtypeStruct((N_NODES, D_OUT), jnp.float32),
    )(q, W2, b2.reshape(1, D_OUT))


def kernel(features, edge_index, W1, b1, W2, b2):
    src = edge_index[0].astype(jnp.int32)
    dst = edge_index[1].astype(jnp.int32)
    n_pad = E_PAD - src.shape[0]
    src_p = jnp.concatenate(
        [src, jnp.zeros((n_pad,), jnp.int32)]).reshape(NW, CH, CHUNK)
    trash = N_NODES + (jnp.arange(n_pad, dtype=jnp.int32) % (N_PAD - N_NODES))
    dst_p = jnp.concatenate([dst, trash]).reshape(NW, CH, CHUNK)

    h = _project_in(features, W1)
    p = _segsum(h, src_p, dst_p)
    x = _bias_relu(p[:, :N_NODES, :], b1)
    q = _segsum(x, src_p, dst_p)
    return _project_out(q[:, :N_NODES, :], W2, b2)


# speedup vs baseline: 11.1054x; 11.1054x over previous
import functools

import jax
import jax.numpy as jnp
from jax import lax
from jax.experimental import pallas as pl
from jax.experimental.pallas import tpu as pltpu
from jax.experimental.pallas import tpu_sc as plsc

N_NODES = 10000
D_IN = 128
D_HID = 16
D_OUT = 128

NC = 2
NS = 16
NW = NC * NS
CHUNK = 128
CH = 80
E_PAD = NW * CH * CHUNK

N_PAD = 10240
ROWS_PER_SUB = N_PAD // NS


def _segsum(h, src_p, dst_p):
    mesh = plsc.VectorSubcoreMesh(core_axis_name="c", subcore_axis_name="s")

    @functools.partial(
        pl.kernel,
        mesh=mesh,
        compiler_params=pltpu.CompilerParams(use_tc_tiling_on_sc=False),
        out_type=jax.ShapeDtypeStruct((NC, N_PAD, D_HID), jnp.float32),
        scratch_types=[
            pltpu.VMEM((CH, CHUNK), jnp.int32),
            pltpu.VMEM((CH, CHUNK), jnp.int32),
            pltpu.VMEM((CHUNK, D_HID), jnp.float32),
            pltpu.VMEM((ROWS_PER_SUB, D_HID), jnp.float32),
            pltpu.VMEM_SHARED((N_PAD, D_HID), jnp.float32),
            pltpu.SemaphoreType.DMA,
        ],
    )
    def segsum_kernel(h_hbm, src_hbm, dst_hbm, out_hbm,
                      src_v, dst_v, msg_v, zero_v, acc_sh, gsem):
        c = lax.axis_index("c")
        s = lax.axis_index("s")
        wid = c * NS + s

        zval = jnp.zeros((D_HID,), jnp.float32)

        @pl.loop(0, ROWS_PER_SUB)
        def _zero(i):
            zero_v[i, :] = zval

        pltpu.sync_copy(zero_v, acc_sh.at[pl.ds(s * ROWS_PER_SUB, ROWS_PER_SUB)])
        plsc.subcore_barrier()

        pltpu.sync_copy(src_hbm.at[wid], src_v)
        pltpu.sync_copy(dst_hbm.at[wid], dst_v)

        @pl.loop(0, CH)
        def _edges(j):
            pltpu.async_copy(h_hbm.at[src_v.at[j]], msg_v, gsem).wait()
            pltpu.sync_copy(msg_v, acc_sh.at[dst_v.at[j]], add=True)

        plsc.subcore_barrier()
        pltpu.sync_copy(
            acc_sh.at[pl.ds(s * ROWS_PER_SUB, ROWS_PER_SUB)],
            out_hbm.at[c, pl.ds(s * ROWS_PER_SUB, ROWS_PER_SUB)],
        )

    return segsum_kernel(h, src_p, dst_p)


def _project_in(features, W1):
    def body(f_ref, w_ref, o_ref):
        o_ref[...] = jnp.dot(f_ref[...], w_ref[...],
                             preferred_element_type=jnp.float32)

    return pl.pallas_call(
        body,
        out_shape=jax.ShapeDtypeStruct((N_NODES, D_HID), jnp.float32),
    )(features, W1)


def _bias_relu(p, b1):
    def body(p_ref, b_ref, o_ref):
        o_ref[...] = jnp.maximum(p_ref[0] + p_ref[1] + b_ref[...], 0.0)

    return pl.pallas_call(
        body,
        out_shape=jax.ShapeDtypeStruct((N_NODES, D_HID), jnp.float32),
    )(p, b1.reshape(1, D_HID))


def _project_out(q, W2, b2):
    def body(q_ref, w_ref, b_ref, o_ref):
        agg = q_ref[0] + q_ref[1]
        o_ref[...] = jnp.dot(agg, w_ref[...],
                             preferred_element_type=jnp.float32) + b_ref[...]

    return pl.pallas_call(
        body,
        out_shape=jax.ShapeDtypeStruct((N_NODES, D_OUT), jnp.float32),
    )(q, W2, b2.reshape(1, D_OUT))


def kernel(features, edge_index, W1, b1, W2, b2):
    src = edge_index[0].astype(jnp.int32)
    dst = edge_index[1].astype(jnp.int32)
    n_pad = E_PAD - src.shape[0]
    src_p = jnp.concatenate(
        [src, jnp.zeros((n_pad,), jnp.int32)]).reshape(NW, CH, CHUNK)
    trash = N_NODES + (jnp.arange(n_pad, dtype=jnp.int32) % (N_PAD - N_NODES))
    dst_p = jnp.concatenate([dst, trash]).reshape(NW, CH, CHUNK)

    h = _project_in(features, W1)
    p = _segsum(h, src_p, dst_p)
    x = _bias_relu(p[:, :N_NODES, :], b1)
    q = _segsum(x, src_p, dst_p)
    return _project_out(q[:, :N_NODES, :], W2, b2)


# speedup vs baseline: 14.8235x; 1.3348x over previous
import functools

import jax
import jax.numpy as jnp
from jax import lax
from jax.experimental import pallas as pl
from jax.experimental.pallas import tpu as pltpu
from jax.experimental.pallas import tpu_sc as plsc

N_NODES = 10000
D_IN = 128
D_HID = 16
D_OUT = 128

NC = 2
NS = 16
NW = NC * NS
CHUNK = 128
CH = 80
E_PAD = NW * CH * CHUNK

N_PAD = 10240
ROWS_PER_SUB = N_PAD // NS
NBUF = 4


def _segsum(h, src_p, dst_p):
    mesh = plsc.VectorSubcoreMesh(core_axis_name="c", subcore_axis_name="s")

    @functools.partial(
        pl.kernel,
        mesh=mesh,
        compiler_params=pltpu.CompilerParams(use_tc_tiling_on_sc=False),
        out_type=jax.ShapeDtypeStruct((NC, N_PAD, D_HID), jnp.float32),
    scratch_types=[
            pltpu.VMEM((CH, CHUNK), jnp.int32),
            pltpu.VMEM((CH, CHUNK), jnp.int32),
            pltpu.VMEM((NBUF, CHUNK, D_HID), jnp.float32),
            pltpu.VMEM((ROWS_PER_SUB, D_HID), jnp.float32),
            pltpu.VMEM_SHARED((N_PAD, D_HID), jnp.float32),
        ] + [pltpu.SemaphoreType.DMA] * (2 * NBUF),
    )
    def segsum_kernel(h_hbm, src_hbm, dst_hbm, out_hbm,
                      src_v, dst_v, msg_v, zero_v, acc_sh, *sems):
        gs = sems[:NBUF]
        ss = sems[NBUF:]
        c = lax.axis_index("c")
        s = lax.axis_index("s")
        wid = c * NS + s

        pltpu.async_copy(src_hbm.at[wid], src_v, gs[0])
        pltpu.async_copy(dst_hbm.at[wid], dst_v, gs[1])

        zval = jnp.zeros((D_HID,), jnp.float32)

        @pl.loop(0, ROWS_PER_SUB)
        def _zero(i):
            zero_v[i, :] = zval

        pltpu.sync_copy(zero_v, acc_sh.at[pl.ds(s * ROWS_PER_SUB, ROWS_PER_SUB)])

        pltpu.make_async_copy(src_hbm.at[wid], src_v, gs[0]).wait()
        pltpu.make_async_copy(dst_hbm.at[wid], dst_v, gs[1]).wait()
        plsc.subcore_barrier()

        def gather(k, b):
            pltpu.async_copy(h_hbm.at[src_v.at[k]], msg_v.at[b], gs[b])

        def wait_gather(b):
            pltpu.make_async_copy(h_hbm.at[src_v.at[0]], msg_v.at[b],
                                  gs[b]).wait()

        def scatter(k, b):
            pltpu.async_copy(msg_v.at[b], acc_sh.at[dst_v.at[k]], ss[b],
                             add=True)

        def wait_scatter(b):
            pltpu.make_async_copy(msg_v.at[b], acc_sh.at[dst_v.at[0]],
                                  ss[b]).wait()

        for b in range(NBUF):
            gather(b, b)

        @pl.loop(0, CH - NBUF, step=NBUF)
        def _edges(j):
            for b in range(NBUF):
                wait_gather(b)
                scatter(j + b, b)
            for b in range(NBUF):
                wait_scatter(b)
                gather(j + NBUF + b, b)

        for b in range(NBUF):
            wait_gather(b)
            scatter(CH - NBUF + b, b)
        for b in range(NBUF):
            wait_scatter(b)

        plsc.subcore_barrier()
        pltpu.sync_copy(
            acc_sh.at[pl.ds(s * ROWS_PER_SUB, ROWS_PER_SUB)],
            out_hbm.at[c, pl.ds(s * ROWS_PER_SUB, ROWS_PER_SUB)],
        )

    return segsum_kernel(h, src_p, dst_p)


def _project_in(features, W1):
    def body(f_ref, w_ref, o_ref):
        o_ref[...] = jnp.dot(f_ref[...], w_ref[...],
                             preferred_element_type=jnp.float32)

    return pl.pallas_call(
        body,
        out_shape=jax.ShapeDtypeStruct((N_NODES, D_HID), jnp.float32),
    )(features, W1)


def _bias_relu(p, b1):
    def body(p_ref, b_ref, o_ref):
        o_ref[...] = jnp.maximum(p_ref[0] + p_ref[1] + b_ref[...], 0.0)

    return pl.pallas_call(
        body,
        out_shape=jax.ShapeDtypeStruct((N_NODES, D_HID), jnp.float32),
    )(p, b1.reshape(1, D_HID))


def _project_out(q, W2, b2):
    def body(q_ref, w_ref, b_ref, o_ref):
        agg = q_ref[0] + q_ref[1]
        o_ref[...] = jnp.dot(agg, w_ref[...],
                             preferred_element_type=jnp.float32) + b_ref[...]

    return pl.pallas_call(
        body,
        out_shape=jax.ShapeDtypeStruct((N_NODES, D_OUT), jnp.float32),
    )(q, W2, b2.reshape(1, D_OUT))


def kernel(features, edge_index, W1, b1, W2, b2):
    src = edge_index[0].astype(jnp.int32)
    dst = edge_index[1].astype(jnp.int32)
    n_pad = E_PAD - src.shape[0]
    src_p = jnp.concatenate(
        [src, jnp.zeros((n_pad,), jnp.int32)]).reshape(NW, CH, CHUNK)
    trash = N_NODES + (jnp.arange(n_pad, dtype=jnp.int32) % (N_PAD - N_NODES))
    dst_p = jnp.concatenate([dst, trash]).reshape(NW, CH, CHUNK)

    h = _project_in(features, W1)
    p = _segsum(h, src_p, dst_p)
    x = _bias_relu(p[:, :N_NODES, :], b1)
    q = _segsum(x, src_p, dst_p)
    return _project_out(q[:, :N_NODES, :], W2, b2)


# speedup vs baseline: 15.0421x; 1.0147x over previous
import functools

import jax
import jax.numpy as jnp
from jax import lax
from jax.experimental import pallas as pl
from jax.experimental.pallas import tpu as pltpu
from jax.experimental.pallas import tpu_sc as plsc

N_NODES = 10000
D_IN = 128
D_HID = 16
D_OUT = 128

NC = 2
NS = 16
NW = NC * NS
CHUNK = 128
CH = 80
E_PAD = NW * CH * CHUNK

N_PAD = 10240
ROWS_PER_SUB = N_PAD // NS
NBUF = 8


def _segsum(h, src_p, dst_p):
    mesh = plsc.VectorSubcoreMesh(core_axis_name="c", subcore_axis_name="s")

    @functools.partial(
        pl.kernel,
        mesh=mesh,
        compiler_params=pltpu.CompilerParams(use_tc_tiling_on_sc=False),
        out_type=jax.ShapeDtypeStruct((NC, N_PAD, D_HID), jnp.float32),
    scratch_types=[
            pltpu.VMEM((CH, CHUNK), jnp.int32),
            pltpu.VMEM((CH, CHUNK), jnp.int32),
            pltpu.VMEM((NBUF, CHUNK, D_HID), jnp.float32),
            pltpu.VMEM((ROWS_PER_SUB, D_HID), jnp.float32),
            pltpu.VMEM_SHARED((N_PAD, D_HID), jnp.float32),
        ] + [pltpu.SemaphoreType.DMA] * (2 * NBUF),
    )
    def segsum_kernel(h_hbm, src_hbm, dst_hbm, out_hbm,
                      src_v, dst_v, msg_v, zero_v, acc_sh, *sems):
        gs = sems[:NBUF]
        ss = sems[NBUF:]
        c = lax.axis_index("c")
        s = lax.axis_index("s")
        wid = c * NS + s

        pltpu.async_copy(src_hbm.at[wid], src_v, gs[0])
        pltpu.async_copy(dst_hbm.at[wid], dst_v, gs[1])

        zval = jnp.zeros((D_HID,), jnp.float32)

        @pl.loop(0, ROWS_PER_SUB)
        def _zero(i):
            zero_v[i, :] = zval

        pltpu.sync_copy(zero_v, acc_sh.at[pl.ds(s * ROWS_PER_SUB, ROWS_PER_SUB)])

        pltpu.make_async_copy(src_hbm.at[wid], src_v, gs[0]).wait()
        pltpu.make_async_copy(dst_hbm.at[wid], dst_v, gs[1]).wait()
        plsc.subcore_barrier()

        def gather(k, b):
            pltpu.async_copy(h_hbm.at[src_v.at[k]], msg_v.at[b], gs[b])

        def wait_gather(b):
            pltpu.make_async_copy(h_hbm.at[src_v.at[0]], msg_v.at[b],
                                  gs[b]).wait()

        def scatter(k, b):
            pltpu.async_copy(msg_v.at[b], acc_sh.at[dst_v.at[k]], ss[b],
                             add=True)

        def wait_scatter(b):
            pltpu.make_async_copy(msg_v.at[b], acc_sh.at[dst_v.at[0]],
                                  ss[b]).wait()

        for b in range(NBUF):
            gather(b, b)

        @pl.loop(0, CH - NBUF, step=NBUF)
        def _edges(j):
            for b in range(NBUF):
                wait_gather(b)
                scatter(j + b, b)
            for b in range(NBUF):
                wait_scatter(b)
                gather(j + NBUF + b, b)

        for b in range(NBUF):
            wait_gather(b)
            scatter(CH - NBUF + b, b)
        for b in range(NBUF):
            wait_scatter(b)

        plsc.subcore_barrier()
        pltpu.sync_copy(
            acc_sh.at[pl.ds(s * ROWS_PER_SUB, ROWS_PER_SUB)],
            out_hbm.at[c, pl.ds(s * ROWS_PER_SUB, ROWS_PER_SUB)],
        )

    return segsum_kernel(h, src_p, dst_p)


def _project_in(features, W1):
    def body(f_ref, w_ref, o_ref):
        o_ref[...] = jnp.dot(f_ref[...], w_ref[...],
                             preferred_element_type=jnp.float32)

    return pl.pallas_call(
        body,
        out_shape=jax.ShapeDtypeStruct((N_NODES, D_HID), jnp.float32),
    )(features, W1)


def _bias_relu(p, b1):
    def body(p_ref, b_ref, o_ref):
        o_ref[...] = jnp.maximum(p_ref[0] + p_ref[1] + b_ref[...], 0.0)

    return pl.pallas_call(
        body,
        out_shape=jax.ShapeDtypeStruct((N_NODES, D_HID), jnp.float32),
    )(p, b1.reshape(1, D_HID))


def _project_out(q, W2, b2):
    def body(q_ref, w_ref, b_ref, o_ref):
        agg = q_ref[0] + q_ref[1]
        o_ref[...] = jnp.dot(agg, w_ref[...],
                             preferred_element_type=jnp.float32) + b_ref[...]

    return pl.pallas_call(
        body,
        out_shape=jax.ShapeDtypeStruct((N_NODES, D_OUT), jnp.float32),
    )(q, W2, b2.reshape(1, D_OUT))


def kernel(features, edge_index, W1, b1, W2, b2):
    src = edge_index[0].astype(jnp.int32)
    dst = edge_index[1].astype(jnp.int32)
    n_pad = E_PAD - src.shape[0]
    src_p = jnp.concatenate(
        [src, jnp.zeros((n_pad,), jnp.int32)]).reshape(NW, CH, CHUNK)
    trash = N_NODES + (jnp.arange(n_pad, dtype=jnp.int32) % (N_PAD - N_NODES))
    dst_p = jnp.concatenate([dst, trash]).reshape(NW, CH, CHUNK)

    h = _project_in(features, W1)
    p = _segsum(h, src_p, dst_p)
    x = _bias_relu(p[:, :N_NODES, :], b1)
    q = _segsum(x, src_p, dst_p)
    return _project_out(q[:, :N_NODES, :], W2, b2)


# speedup vs baseline: 26.6482x; 1.7716x over previous
import functools

import jax
import jax.numpy as jnp
from jax import lax
from jax.experimental import pallas as pl
from jax.experimental.pallas import tpu as pltpu
from jax.experimental.pallas import tpu_sc as plsc

N_NODES = 10000
N_EDGES = 320000
D_IN = 128
D_HID = 16
D_OUT = 128

NC = 2
NS = 16
NW = NC * NS
CHUNK = 128
N_CHUNKS = N_EDGES // CHUNK
CH = N_CHUNKS // NW
N_REM = N_CHUNKS - CH * NW

N_PAD = 10240
ROWS_PER_SUB = N_PAD // NS
NBUF = 6


def _segsum(h, src2d, dst2d):
    mesh = plsc.VectorSubcoreMesh(core_axis_name="c", subcore_axis_name="s")

    @functools.partial(
        pl.kernel,
        mesh=mesh,
        compiler_params=pltpu.CompilerParams(use_tc_tiling_on_sc=False),
        out_type=jax.ShapeDtypeStruct((NC, N_PAD, D_HID), jnp.float32),
        scratch_types=[
            pltpu.VMEM((CH + 1, CHUNK), jnp.int32),
            pltpu.VMEM((CH + 1, CHUNK), jnp.int32),
            pltpu.VMEM((NBUF, CHUNK, D_HID), jnp.float32),
            pltpu.VMEM((ROWS_PER_SUB, D_HID), jnp.float32),
            pltpu.VMEM_SHARED((N_PAD, D_HID), jnp.float32),
        ] + [pltpu.SemaphoreType.DMA] * (2 * NBUF),
    )
    def segsum_kernel(h_hbm, src_hbm, dst_hbm, out_hbm,
                      src_v, dst_v, msg_v, zero_v, acc_sh, *sems):
        gs = sems[:NBUF]
        ss = sems[NBUF:]
        c = lax.axis_index("c")
        s = lax.axis_index("s")
        wid = c * NS + s

        pltpu.async_copy(src_hbm.at[pl.ds(wid * CH, CH)],
                         src_v.at[pl.ds(0, CH)], gs[0])
        pltpu.async_copy(dst_hbm.at[pl.ds(wid * CH, CH)],
                         dst_v.at[pl.ds(0, CH)], gs[1])

        zval = jnp.zeros((D_HID,), jnp.float32)

        @pl.loop(0, ROWS_PER_SUB)
        def _zero(i):
            zero_v[i, :] = zval

        pltpu.sync_copy(zero_v, acc_sh.at[pl.ds(s * ROWS_PER_SUB, ROWS_PER_SUB)])

        pltpu.make_async_copy(src_hbm.at[pl.ds(0, CH)],
                              src_v.at[pl.ds(0, CH)], gs[0]).wait()
        pltpu.make_async_copy(dst_hbm.at[pl.ds(0, CH)],
                              dst_v.at[pl.ds(0, CH)], gs[1]).wait()

        @pl.when(wid < N_REM)
        def _rem_idx():
            pltpu.sync_copy(src_hbm.at[pl.ds(NW * CH + wid, 1)],
                            src_v.at[pl.ds(CH, 1)])
            pltpu.sync_copy(dst_hbm.at[pl.ds(NW * CH + wid, 1)],
                            dst_v.at[pl.ds(CH, 1)])

        plsc.subcore_barrier()

        def gather(k, b):
            pltpu.async_copy(h_hbm.at[src_v.at[k]], msg_v.at[b], gs[b])

        def wait_gather(b):
            pltpu.make_async_copy(h_hbm.at[src_v.at[0]], msg_v.at[b],
                                  gs[b]).wait()

        def scatter(k, b):
            pltpu.async_copy(msg_v.at[b], acc_sh.at[dst_v.at[k]], ss[b],
                             add=True)

        def wait_scatter(b):
            pltpu.make_async_copy(msg_v.at[b], acc_sh.at[dst_v.at[0]],
                                  ss[b]).wait()

        for b in range(NBUF):
            gather(b, b)

        @pl.loop(0, CH - NBUF, step=NBUF)
        def _edges(j):
            for b in range(NBUF):
                wait_gather(b)
                scatter(j + b, b)
            for b in range(NBUF):
                wait_scatter(b)
                gather(j + NBUF + b, b)

        for b in range(NBUF):
            wait_gather(b)
            scatter(CH - NBUF + b, b)
        for b in range(NBUF):
            wait_scatter(b)

        @pl.when(wid < N_REM)
        def _rem_edges():
            gather(CH, 0)
            wait_gather(0)
            scatter(CH, 0)
            wait_scatter(0)

        plsc.subcore_barrier()
        pltpu.sync_copy(
            acc_sh.at[pl.ds(s * ROWS_PER_SUB, ROWS_PER_SUB)],
            out_hbm.at[c, pl.ds(s * ROWS_PER_SUB, ROWS_PER_SUB)],
        )

    return segsum_kernel(h, src2d, dst2d)


def _project_in(features, W1):
    def body(f_ref, w_ref, o_ref):
        o_ref[...] = jnp.dot(f_ref[...], w_ref[...],
                             preferred_element_type=jnp.float32)

    return pl.pallas_call(
        body,
        out_shape=jax.ShapeDtypeStruct((N_NODES, D_HID), jnp.float32),
    )(features, W1)


def _bias_relu(p, b1):
    def body(p_ref, b_ref, o_ref):
        o_ref[...] = jnp.maximum(p_ref[0] + p_ref[1] + b_ref[...], 0.0)

    return pl.pallas_call(
        body,
        out_shape=jax.ShapeDtypeStruct((N_PAD, D_HID), jnp.float32),
    )(p, b1.reshape(1, D_HID))


def _project_out(q, W2, b2):
    def body(q_ref, w_ref, b_ref, o_ref):
        agg = q_ref[0, :N_NODES, :] + q_ref[1, :N_NODES, :]
        o_ref[...] = jnp.dot(agg, w_ref[...],
                             preferred_element_type=jnp.float32) + b_ref[...]

    return pl.pallas_call(
        body,
        out_shape=jax.ShapeDtypeStruct((N_NODES, D_OUT), jnp.float32),
    )(q, W2, b2.reshape(1, D_OUT))


def kernel(features, edge_index, W1, b1, W2, b2):
    src2d = edge_index[0].astype(jnp.int32).reshape(N_CHUNKS, CHUNK)
    dst2d = edge_index[1].astype(jnp.int32).reshape(N_CHUNKS, CHUNK)

    h = _project_in(features, W1)
    p = _segsum(h, src2d, dst2d)
    x = _bias_relu(p, b1)
    q = _segsum(x, src2d, dst2d)
    return _project_out(q, W2, b2)


# speedup vs baseline: 28.9667x; 1.0870x over previous
import functools

import jax
import jax.numpy as jnp
from jax import lax
from jax.experimental import pallas as pl
from jax.experimental.pallas import tpu as pltpu
from jax.experimental.pallas import tpu_sc as plsc

N_NODES = 10000
N_EDGES = 320000
D_IN = 128
D_HID = 16
D_OUT = 128

NC = 2
NS = 16
NW = NC * NS
CHUNK = 128
N_CHUNKS = N_EDGES // CHUNK
CH = N_CHUNKS // NW
N_REM = N_CHUNKS - CH * NW

N_PAD = 10240
ROWS_PER_SUB = N_PAD // NS
NBUF = 6


def _segsum(h, edges):
    mesh = plsc.VectorSubcoreMesh(core_axis_name="c", subcore_axis_name="s")

    @functools.partial(
        pl.kernel,
        mesh=mesh,
        compiler_params=pltpu.CompilerParams(use_tc_tiling_on_sc=False),
        out_type=jax.ShapeDtypeStruct((NC, N_PAD, D_HID), jnp.float32),
        scratch_types=[
            pltpu.VMEM((CH + 1, CHUNK), jnp.int32),
            pltpu.VMEM((CH + 1, CHUNK), jnp.int32),
            pltpu.VMEM((NBUF, CHUNK, D_HID), jnp.float32),
            pltpu.VMEM((ROWS_PER_SUB, D_HID), jnp.float32),
            pltpu.VMEM_SHARED((N_PAD, D_HID), jnp.float32),
        ] + [pltpu.SemaphoreType.DMA] * (2 * NBUF),
    )
    def segsum_kernel(h_hbm, e_hbm, out_hbm,
                      src_v, dst_v, msg_v, zero_v, acc_sh, *sems):
        gs = sems[:NBUF]
        ss = sems[NBUF:]
        c = lax.axis_index("c")
        s = lax.axis_index("s")
        wid = c * NS + s

        pltpu.async_copy(e_hbm.at[0, pl.ds(wid * CH, CH)],
                         src_v.at[pl.ds(0, CH)], gs[0])
        pltpu.async_copy(e_hbm.at[1, pl.ds(wid * CH, CH)],
                         dst_v.at[pl.ds(0, CH)], gs[1])

        zval = jnp.zeros((D_HID,), jnp.float32)

        @pl.loop(0, ROWS_PER_SUB)
        def _zero(i):
            zero_v[i, :] = zval

        pltpu.sync_copy(zero_v, acc_sh.at[pl.ds(s * ROWS_PER_SUB, ROWS_PER_SUB)])

        pltpu.make_async_copy(e_hbm.at[0, pl.ds(0, CH)],
                              src_v.at[pl.ds(0, CH)], gs[0]).wait()
        pltpu.make_async_copy(e_hbm.at[1, pl.ds(0, CH)],
                              dst_v.at[pl.ds(0, CH)], gs[1]).wait()

        @pl.when(wid < N_REM)
        def _rem_idx():
            pltpu.sync_copy(e_hbm.at[0, pl.ds(NW * CH + wid, 1)],
                            src_v.at[pl.ds(CH, 1)])
            pltpu.sync_copy(e_hbm.at[1, pl.ds(NW * CH + wid, 1)],
                            dst_v.at[pl.ds(CH, 1)])

        plsc.subcore_barrier()

        def gather(k, b):
            pltpu.async_copy(h_hbm.at[src_v.at[k]], msg_v.at[b], gs[b])

        def wait_gather(b):
            pltpu.make_async_copy(h_hbm.at[src_v.at[0]], msg_v.at[b],
                                  gs[b]).wait()

        def scatter(k, b):
            pltpu.async_copy(msg_v.at[b], acc_sh.at[dst_v.at[k]], ss[b],
                             add=True)

        def wait_scatter(b):
            pltpu.make_async_copy(msg_v.at[b], acc_sh.at[dst_v.at[0]],
                                  ss[b]).wait()

        for b in range(NBUF):
            gather(b, b)

        @pl.loop(0, CH - NBUF, step=NBUF)
        def _edges(j):
            for b in range(NBUF):
                wait_gather(b)
                scatter(j + b, b)
            for b in range(NBUF):
                wait_scatter(b)
                gather(j + NBUF + b, b)

        for b in range(NBUF):
            wait_gather(b)
            scatter(CH - NBUF + b, b)
        for b in range(NBUF):
            wait_scatter(b)

        @pl.when(wid < N_REM)
        def _rem_edges():
            gather(CH, 0)
            wait_gather(0)
            scatter(CH, 0)
            wait_scatter(0)

        plsc.subcore_barrier()
        pltpu.sync_copy(
            acc_sh.at[pl.ds(s * ROWS_PER_SUB, ROWS_PER_SUB)],
            out_hbm.at[c, pl.ds(s * ROWS_PER_SUB, ROWS_PER_SUB)],
        )

    return segsum_kernel(h, edges)


def _project_in(features, W1):
    def body(f_ref, w_ref, o_ref):
        o_ref[...] = jnp.dot(f_ref[...], w_ref[...],
                             preferred_element_type=jnp.float32)

    return pl.pallas_call(
        body,
        out_shape=jax.ShapeDtypeStruct((N_NODES, D_HID), jnp.float32),
    )(features, W1)


def _bias_relu(p, b1):
    def body(p_ref, b_ref, o_ref):
        o_ref[...] = jnp.maximum(p_ref[0] + p_ref[1] + b_ref[...], 0.0)

    return pl.pallas_call(
        body,
        out_shape=jax.ShapeDtypeStruct((N_PAD, D_HID), jnp.float32),
    )(p, b1.reshape(1, D_HID))


def _project_out(q, W2, b2):
    def body(q_ref, w_ref, b_ref, o_ref):
        agg = q_ref[0, :N_NODES, :] + q_ref[1, :N_NODES, :]
        o_ref[...] = jnp.dot(agg, w_ref[...],
                             preferred_element_type=jnp.float32) + b_ref[...]

    return pl.pallas_call(
        body,
        out_shape=jax.ShapeDtypeStruct((N_NODES, D_OUT), jnp.float32),
    )(q, W2, b2.reshape(1, D_OUT))


def kernel(features, edge_index, W1, b1, W2, b2):
    edges = edge_index.astype(jnp.int32).reshape(2, N_CHUNKS, CHUNK)

    h = _project_in(features, W1)
    p = _segsum(h, edges)
    x = _bias_relu(p, b1)
    q = _segsum(x, edges)
    return _project_out(q, W2, b2)


# speedup vs baseline: 37.9692x; 1.3108x over previous
import functools

import jax
import jax.numpy as jnp
from jax import lax
from jax.experimental import pallas as pl
from jax.experimental.pallas import tpu as pltpu
from jax.experimental.pallas import tpu_sc as plsc

N_NODES = 10000
N_EDGES = 320000
D_IN = 128
D_HID = 16
D_OUT = 128

NC = 2
NS = 16
NW = NC * NS
CHUNK = 128
N_CHUNKS = N_EDGES // CHUNK
CH = N_CHUNKS // NW
N_REM = N_CHUNKS - CH * NW

N_PAD = 10240
ROWS_PER_SUB = N_PAD // NS
NBUF = 6


def _segsum(h, edges):
    mesh = plsc.VectorSubcoreMesh(core_axis_name="c", subcore_axis_name="s")

    @functools.partial(
        pl.kernel,
        mesh=mesh,
        compiler_params=pltpu.CompilerParams(use_tc_tiling_on_sc=False),
        out_type=jax.ShapeDtypeStruct((NC, N_PAD, D_HID), jnp.float32),
        scratch_types=[
            pltpu.VMEM((CH + 1, CHUNK), jnp.int32),
            pltpu.VMEM((CH + 1, CHUNK), jnp.int32),
            pltpu.VMEM((NBUF, CHUNK, D_HID), jnp.float32),
            pltpu.VMEM((ROWS_PER_SUB, D_HID), jnp.float32),
            pltpu.VMEM_SHARED((N_PAD, D_HID), jnp.float32),
        ] + [pltpu.SemaphoreType.DMA] * (2 * NBUF),
    )
    def segsum_kernel(h_hbm, e_hbm, out_hbm,
                      src_v, dst_v, msg_v, zero_v, acc_sh, *sems):
        gs = sems[:NBUF]
        ss = sems[NBUF:]
        c = lax.axis_index("c")
        s = lax.axis_index("s")
        wid = c * NS + s

        pltpu.async_copy(e_hbm.at[0, pl.ds(wid * CH, CH)],
                         src_v.at[pl.ds(0, CH)], gs[0])
        pltpu.async_copy(e_hbm.at[1, pl.ds(wid * CH, CH)],
                         dst_v.at[pl.ds(0, CH)], gs[1])

        zval = jnp.zeros((D_HID,), jnp.float32)

        @pl.loop(0, ROWS_PER_SUB)
        def _zero(i):
            zero_v[i, :] = zval

        pltpu.sync_copy(zero_v, acc_sh.at[pl.ds(s * ROWS_PER_SUB, ROWS_PER_SUB)])

        pltpu.make_async_copy(e_hbm.at[0, pl.ds(0, CH)],
                              src_v.at[pl.ds(0, CH)], gs[0]).wait()
        pltpu.make_async_copy(e_hbm.at[1, pl.ds(0, CH)],
                              dst_v.at[pl.ds(0, CH)], gs[1]).wait()

        @pl.when(wid < N_REM)
        def _rem_idx():
            pltpu.sync_copy(e_hbm.at[0, pl.ds(NW * CH + wid, 1)],
                            src_v.at[pl.ds(CH, 1)])
            pltpu.sync_copy(e_hbm.at[1, pl.ds(NW * CH + wid, 1)],
                            dst_v.at[pl.ds(CH, 1)])

        plsc.subcore_barrier()

        def gather(k, b):
            pltpu.async_copy(h_hbm.at[src_v.at[k]], msg_v.at[b], gs[b])

        def wait_gather(b):
            pltpu.make_async_copy(h_hbm.at[src_v.at[0]], msg_v.at[b],
                                  gs[b]).wait()

        def scatter(k, b):
            pltpu.async_copy(msg_v.at[b], acc_sh.at[dst_v.at[k]], ss[b],
                             add=True)

        def wait_scatter(b):
            pltpu.make_async_copy(msg_v.at[b], acc_sh.at[dst_v.at[0]],
                                  ss[b]).wait()

        for b in range(NBUF):
            gather(b, b)

        @pl.loop(0, CH - NBUF, step=NBUF)
        def _edges(j):
            for b in range(NBUF):
                wait_gather(b)
                scatter(j + b, b)
            for b in range(NBUF):
                wait_scatter(b)
                gather(j + NBUF + b, b)

        for b in range(NBUF):
            wait_gather(b)
            scatter(CH - NBUF + b, b)
        for b in range(NBUF):
            wait_scatter(b)

        @pl.when(wid < N_REM)
        def _rem_edges():
            gather(CH, 0)
            wait_gather(0)
            scatter(CH, 0)
            wait_scatter(0)

        plsc.subcore_barrier()
        pltpu.sync_copy(
            acc_sh.at[pl.ds(s * ROWS_PER_SUB, ROWS_PER_SUB)],
            out_hbm.at[c, pl.ds(s * ROWS_PER_SUB, ROWS_PER_SUB)],
        )

    return segsum_kernel(h, edges)


def _project_in(features, W1):
    rows = N_NODES * D_HID // 128

    def body(f_ref, w_ref, o_ref):
        f2 = jnp.concatenate([f_ref[:, a, :] for a in range(8)], axis=1)
        o_ref[...] = jnp.dot(f2, w_ref[...],
                             preferred_element_type=jnp.float32)

    w_blk = jnp.kron(jnp.eye(8, dtype=jnp.float32), W1)
    return pl.pallas_call(
        body,
        out_shape=jax.ShapeDtypeStruct((rows, 128), jnp.float32),
    )(features.reshape(rows, 8, D_IN), w_blk).reshape(N_NODES, D_HID)


def _bias_relu(p, b1):
    rows = N_PAD * D_HID // 128

    def body(p_ref, b_ref, o_ref):
        o_ref[...] = jnp.maximum(p_ref[0] + p_ref[1] + b_ref[...], 0.0)

    b_wide = jnp.tile(b1, 128 // D_HID).reshape(1, 128)
    return pl.pallas_call(
        body,
        out_shape=jax.ShapeDtypeStruct((rows, 128), jnp.float32),
    )(p.reshape(NC, rows, 128), b_wide).reshape(N_PAD, D_HID)


def _project_out(q, W2, b2):
    rows = N_PAD * D_HID // 128
    out_r = N_NODES // 8

    def body(q_ref, w_ref, b_ref, o_ref):
        agg2 = q_ref[0, :out_r, :] + q_ref[1, :out_r, :]
        out_flat = jnp.dot(agg2, w_ref[...],
                           preferred_element_type=jnp.float32)
        for a in range(8):
            o_ref[:, a, :] = (out_flat[:, 128 * a:128 * (a + 1)]
                              + b_ref[...])

    w_blk = jnp.kron(jnp.eye(8, dtype=jnp.float32), W2)
    return pl.pallas_call(
        body,
        out_shape=jax.ShapeDtypeStruct((out_r, 8, D_OUT), jnp.float32),
    )(q.reshape(NC, rows, 128), w_blk,
      b2.reshape(1, D_OUT)).reshape(N_NODES, D_OUT)


def kernel(features, edge_index, W1, b1, W2, b2):
    edges = edge_index.astype(jnp.int32).reshape(2, N_CHUNKS, CHUNK)

    h = _project_in(features, W1)
    p = _segsum(h, edges)
    x = _bias_relu(p, b1)
    q = _segsum(x, edges)
    return _project_out(q, W2, b2)


# speedup vs baseline: 39.8506x; 1.0496x over previous
import functools

import jax
import jax.numpy as jnp
from jax import lax
from jax.experimental import pallas as pl
from jax.experimental.pallas import tpu as pltpu
from jax.experimental.pallas import tpu_sc as plsc

N_NODES = 10000
N_EDGES = 320000
D_IN = 128
D_HID = 16
D_OUT = 128

NC = 2
NS = 16
NW = NC * NS
CHUNK = 128
N_CHUNKS = N_EDGES // CHUNK
CH = N_CHUNKS // NW
N_REM = N_CHUNKS - CH * NW

N_PAD = 10240
ROWS_PER_SUB = N_PAD // NS
MAC = 13
NMAC = CH // MAC
NBUF = 3


def _segsum(h, e_main, e_rem):
    mesh = plsc.VectorSubcoreMesh(core_axis_name="c", subcore_axis_name="s")
    MB = MAC * CHUNK

    @functools.partial(
        pl.kernel,
        mesh=mesh,
        compiler_params=pltpu.CompilerParams(use_tc_tiling_on_sc=False),
        out_type=jax.ShapeDtypeStruct((NC, N_PAD, D_HID), jnp.float32),
        scratch_types=[
            pltpu.VMEM((NMAC, MB), jnp.int32),
            pltpu.VMEM((NMAC, MB), jnp.int32),
            pltpu.VMEM((2, CHUNK), jnp.int32),
            pltpu.VMEM((NBUF, MB, D_HID), jnp.float32),
            pltpu.VMEM((ROWS_PER_SUB, D_HID), jnp.float32),
            pltpu.VMEM_SHARED((N_PAD, D_HID), jnp.float32),
        ] + [pltpu.SemaphoreType.DMA] * (2 * NBUF),
    )
    def segsum_kernel(h_hbm, e_hbm, er_hbm, out_hbm,
                      src_v, dst_v, rem_v, msg_v, zero_v, acc_sh, *sems):
        gs = sems[:NBUF]
        ss = sems[NBUF:]
        c = lax.axis_index("c")
        s = lax.axis_index("s")
        wid = c * NS + s

        pltpu.async_copy(e_hbm.at[0, wid], src_v, gs[0])
        pltpu.async_copy(e_hbm.at[1, wid], dst_v, ss[0])

        zval = jnp.zeros((D_HID,), jnp.float32)

        @pl.loop(0, ROWS_PER_SUB)
        def _zero(i):
            zero_v[i, :] = zval

        def gather(m, b):
            pltpu.async_copy(h_hbm.at[src_v.at[m]], msg_v.at[b], gs[b])

        def wait_gather(b):
            pltpu.make_async_copy(h_hbm.at[src_v.at[0]], msg_v.at[b],
                                  gs[b]).wait()

        def scatter(m, b):
            pltpu.async_copy(msg_v.at[b], acc_sh.at[dst_v.at[m]],
                             ss[b], add=True)

        def wait_scatter(b):
            pltpu.make_async_copy(msg_v.at[b], acc_sh.at[dst_v.at[0]],
                                  ss[b]).wait()

        pltpu.make_async_copy(e_hbm.at[0, wid], src_v, gs[0]).wait()
        for b in range(NBUF):
            gather(b, b)

        pltpu.sync_copy(zero_v, acc_sh.at[pl.ds(s * ROWS_PER_SUB, ROWS_PER_SUB)])

        pltpu.make_async_copy(e_hbm.at[1, wid], dst_v, ss[0]).wait()

        @pl.when(wid < N_REM)
        def _rem_idx():
            pltpu.sync_copy(er_hbm.at[0, pl.ds(wid, 1)],
                            rem_v.at[pl.ds(0, 1)])
            pltpu.sync_copy(er_hbm.at[1, pl.ds(wid, 1)],
                            rem_v.at[pl.ds(1, 1)])

        plsc.subcore_barrier()

        for m in range(NMAC):
            b = m % NBUF
            wait_gather(b)
            scatter(m, b)
            if m + NBUF < NMAC:
                wait_scatter(b)
                gather(m + NBUF, b)
        for m in range(NMAC - NBUF, NMAC):
            wait_scatter(m % NBUF)

        @pl.when(wid < N_REM)
        def _rem_edges():
            pltpu.sync_copy(h_hbm.at[rem_v.at[0]],
                            msg_v.at[0, pl.ds(0, CHUNK)])
            pltpu.sync_copy(msg_v.at[0, pl.ds(0, CHUNK)],
                            acc_sh.at[rem_v.at[1]], add=True)

        plsc.subcore_barrier()
        pltpu.sync_copy(
            acc_sh.at[pl.ds(s * ROWS_PER_SUB, ROWS_PER_SUB)],
            out_hbm.at[c, pl.ds(s * ROWS_PER_SUB, ROWS_PER_SUB)],
        )

    return segsum_kernel(h, e_main, e_rem)


def _project_in(features, W1):
    rows = N_NODES * D_HID // 128

    def body(f_ref, w_ref, o_ref):
        f2 = jnp.concatenate([f_ref[:, a, :] for a in range(8)], axis=1)
        o_ref[...] = jnp.dot(f2, w_ref[...],
                             preferred_element_type=jnp.float32)

    w_blk = jnp.kron(jnp.eye(8, dtype=jnp.float32), W1)
    return pl.pallas_call(
        body,
        out_shape=jax.ShapeDtypeStruct((rows, 128), jnp.float32),
    )(features.reshape(rows, 8, D_IN), w_blk).reshape(N_NODES, D_HID)


def _bias_relu(p, b1):
    rows = N_PAD * D_HID // 128

    def body(p_ref, b_ref, o_ref):
        o_ref[...] = jnp.maximum(p_ref[0] + p_ref[1] + b_ref[...], 0.0)

    b_wide = jnp.tile(b1, 128 // D_HID).reshape(1, 128)
    return pl.pallas_call(
        body,
        out_shape=jax.ShapeDtypeStruct((rows, 128), jnp.float32),
    )(p.reshape(NC, rows, 128), b_wide).reshape(N_PAD, D_HID)


def _project_out(q, W2, b2):
    rows = N_PAD * D_HID // 128
    out_r = N_NODES // 8

    def body(q_ref, w_ref, b_ref, o_ref):
        agg2 = q_ref[0, :out_r, :] + q_ref[1, :out_r, :]
        out_flat = jnp.dot(agg2, w_ref[...],
                           preferred_element_type=jnp.float32)
        for a in range(8):
            o_ref[:, a, :] = (out_flat[:, 128 * a:128 * (a + 1)]
                              + b_ref[...])

    w_blk = jnp.kron(jnp.eye(8, dtype=jnp.float32), W2)
    return pl.pallas_call(
        body,
        out_shape=jax.ShapeDtypeStruct((out_r, 8, D_OUT), jnp.float32),
    )(q.reshape(NC, rows, 128), w_blk,
      b2.reshape(1, D_OUT)).reshape(N_NODES, D_OUT)


def kernel(features, edge_index, W1, b1, W2, b2):
    ef = edge_index.astype(jnp.int32)
    n_main = NW * CH * CHUNK
    e_main = ef[:, :n_main].reshape(2, NW, NMAC, MAC * CHUNK)
    e_rem = ef[:, n_main:].reshape(2, N_REM, CHUNK)

    h = _project_in(features, W1)
    p = _segsum(h, e_main, e_rem)
    x = _bias_relu(p, b1)
    q = _segsum(x, e_main, e_rem)
    return _project_out(q, W2, b2)
